# Initial kernel scaffold; baseline (speedup 1.0000x reference)
import jax
import jax.numpy as jnp
from jax.experimental import pallas as pl


def kernel(predictions, targets):
    raise NotImplementedError("write your pallas kernel here")


# speedup vs baseline: 28.8748x; 28.8748x over previous
import functools

import jax
import jax.numpy as jnp
import numpy as np
from jax import lax
from jax.experimental import pallas as pl
from jax.experimental.pallas import tpu as pltpu
from jax.experimental.pallas import tpu_sc as plsc

N = 3276800
NTILES = 32
PER_TILE = N // NTILES
PIECE = 4096
NPIECES = PER_TILE // PIECE

BP_BITS = 14
BP = 1 << BP_BITS
BT = 1 << 15
CP = BP // 16
CT = BT // 16

LN2 = 0.6931471805599453

_f = 1.0 / np.log2(np.arange(N, dtype=np.float64) + 2.0)
_phi = np.zeros(N + 16, dtype=np.float64)
_phi[1:N + 1] = np.cumsum(_f)
_phi[N + 1:] = _phi[N]
_PHI = _phi.astype(np.float32)

_mesh = plsc.VectorSubcoreMesh(core_axis_name="c", subcore_axis_name="s")

f32 = jnp.float32
i32 = jnp.int32


def _zero_ref(ref, size):
    z = jnp.zeros((16,), f32)

    def body(i, _):
        ref[pl.ds(i * 16, 16)] = z
        return 0

    lax.fori_loop(0, size // 16, body, 0, unroll=4)


@functools.partial(
    pl.kernel,
    out_type=(
        jax.ShapeDtypeStruct((NTILES, BP), f32),
        jax.ShapeDtypeStruct((NTILES, BP), f32),
        jax.ShapeDtypeStruct((NTILES, BT), f32),
        jax.ShapeDtypeStruct((NTILES, BT), f32),
    ),
    mesh=_mesh,
    scratch_types=[
        pltpu.VMEM((BP,), f32),
        pltpu.VMEM((BP,), f32),
        pltpu.VMEM((BT,), f32),
        pltpu.VMEM((BT,), f32),
        pltpu.VMEM((PIECE,), i32),
        pltpu.VMEM((PIECE,), f32),
    ],
    compiler_params=pltpu.CompilerParams(needs_layout_passes=False),
)
def _hist_kernel(pbits, targs, ocp, ogp, oct_, ogt, hcp, hgp, hct, hgt,
                 pbuf, tbuf):
    cid = lax.axis_index("c")
    sid = lax.axis_index("s")
    wid = sid * 2 + cid
    base = wid * PER_TILE

    _zero_ref(hcp, BP)
    _zero_ref(hgp, BP)
    _zero_ref(hct, BT)
    _zero_ref(hgt, BT)

    ones = jnp.ones((16,), f32)

    def piece_body(pi, _):
        off = base + pi * PIECE
        pltpu.sync_copy(pbits.at[pl.ds(off, PIECE)], pbuf)
        pltpu.sync_copy(targs.at[pl.ds(off, PIECE)], tbuf)

        def vec_body(vi, _):
            u = pbuf[pl.ds(vi * 16, 16)]
            t = tbuf[pl.ds(vi * 16, 16)]
            dkey = jnp.where(u < 0, u,
                             jnp.bitwise_and(jnp.bitwise_not(u),
                                             jnp.int32(0x7FFFFFFF)))
            bp_idx = lax.shift_right_logical(dkey, 32 - BP_BITS)
            ti = jnp.clip((t * f32(BT)).astype(i32), 0, BT - 1)
            bt_idx = (BT - 1) - ti
            g = jnp.exp(t * f32(LN2)) - 1.0
            plsc.addupdate_scatter(hcp, [bp_idx], ones)
            plsc.addupdate_scatter(hgp, [bp_idx], g)
            plsc.addupdate_scatter(hct, [bt_idx], ones)
            plsc.addupdate_scatter(hgt, [bt_idx], g)
            return 0

        lax.fori_loop(0, PIECE // 16, vec_body, 0)
        return 0

    lax.fori_loop(0, NPIECES, piece_body, 0)

    pltpu.sync_copy(hcp, ocp.at[wid])
    pltpu.sync_copy(hgp, ogp.at[wid])
    pltpu.sync_copy(hct, oct_.at[wid])
    pltpu.sync_copy(hgt, ogt.at[wid])


def _reduce_rows(buf, acc, width):
    def body(vi, _):
        s = jnp.zeros((16,), f32)
        for r in range(NTILES):
            s = s + buf[r, pl.ds(vi * 16, 16)]
        acc[pl.ds(vi * 16, 16)] = s
        return 0

    lax.fori_loop(0, width // 16, body, 0)


def _vec_total(acc, width):
    def body(vi, s):
        return s + acc[pl.ds(vi * 16, 16)]

    return lax.fori_loop(0, width // 16, body, jnp.zeros((16,), f32))


def _prefix_and_index(acc_c, sidx_a, sidx_b, width, off):
    def body(vi, carry):
        c = acc_c[pl.ds(vi * 16, 16)]
        inc = plsc.cumsum(c)
        excl = inc - c + carry
        sidx_a[pl.ds(vi * 16, 16)] = excl.astype(i32)
        sidx_b[pl.ds(vi * 16, 16)] = (excl + c).astype(i32)
        return carry + jnp.sum(c)

    lax.fori_loop(0, width // 16, body, off)


def _gather_phi(phi, sidx, dst, width, sem):
    copies = []
    for j in range(width // 128):
        copies.append(pltpu.async_copy(
            phi.at[sidx.at[pl.ds(j * 128, 128)]],
            dst.at[pl.ds(j * 128, 128)], sem))
    for c in copies:
        c.wait()


def _dcg_accum(acc_c, acc_g, phi_a, phi_b, width):
    def body(vi, s):
        sl = pl.ds(vi * 16, 16)
        c = acc_c[sl]
        g = acc_g[sl]
        w = (phi_b[sl] - phi_a[sl]) / jnp.maximum(c, 1.0)
        return s + g * w

    return lax.fori_loop(0, width // 16, body, jnp.zeros((16,), f32))


@functools.partial(
    pl.kernel,
    out_type=(
        jax.ShapeDtypeStruct((BP,), f32),
        jax.ShapeDtypeStruct((BP,), f32),
        jax.ShapeDtypeStruct((BT,), f32),
        jax.ShapeDtypeStruct((BT,), f32),
        jax.ShapeDtypeStruct((16, 32), f32),
    ),
    mesh=_mesh,
    scratch_types=[
        pltpu.VMEM((NTILES, CP), f32),
        pltpu.VMEM((NTILES, CT), f32),
        pltpu.VMEM((CP,), f32),
        pltpu.VMEM((CP,), f32),
        pltpu.VMEM((CT,), f32),
        pltpu.VMEM((CT,), f32),
        pltpu.VMEM((32,), f32),
    ],
    compiler_params=pltpu.CompilerParams(needs_layout_passes=False),
)
def _merge_kernel(hcp, hgp, hct, hgt, ocp, ogp, oct_, ogt, otot,
                  buf_p, buf_t, czp, gzp, czt, gzt, pub):
    cid = lax.axis_index("c")
    sid = lax.axis_index("s")

    @pl.when(cid == 0)
    def _stage():
        pltpu.sync_copy(hcp.at[:, pl.ds(sid * CP, CP)], buf_p)
        _reduce_rows(buf_p, czp, CP)
        pltpu.sync_copy(hgp.at[:, pl.ds(sid * CP, CP)], buf_p)
        _reduce_rows(buf_p, gzp, CP)
        pltpu.sync_copy(hct.at[:, pl.ds(sid * CT, CT)], buf_t)
        _reduce_rows(buf_t, czt, CT)
        pltpu.sync_copy(hgt.at[:, pl.ds(sid * CT, CT)], buf_t)
        _reduce_rows(buf_t, gzt, CT)
        pltpu.sync_copy(czp, ocp.at[pl.ds(sid * CP, CP)])
        pltpu.sync_copy(gzp, ogp.at[pl.ds(sid * CP, CP)])
        pltpu.sync_copy(czt, oct_.at[pl.ds(sid * CT, CT)])
        pltpu.sync_copy(gzt, ogt.at[pl.ds(sid * CT, CT)])
        pub[pl.ds(0, 16)] = _vec_total(czp, CP)
        pub[pl.ds(16, 16)] = _vec_total(czt, CT)
        pltpu.sync_copy(pub, otot.at[sid])


@functools.partial(
    pl.kernel,
    out_type=(
        jax.ShapeDtypeStruct((16, 16), f32),
        jax.ShapeDtypeStruct((16, 16), f32),
    ),
    mesh=_mesh,
    scratch_types=[
        pltpu.VMEM((CP,), f32),
        pltpu.VMEM((CP,), f32),
        pltpu.VMEM((CT,), f32),
        pltpu.VMEM((CT,), f32),
        pltpu.VMEM((CT,), i32),
        pltpu.VMEM((CT,), i32),
        pltpu.VMEM((CT,), f32),
        pltpu.VMEM((CT,), f32),
        pltpu.VMEM((16, 32), f32),
        pltpu.VMEM((16,), f32),
        pltpu.SemaphoreType.DMA,
    ],
    compiler_params=pltpu.CompilerParams(needs_layout_passes=False),
)
def _rank_kernel(cph, gph, cth, gth, tot, phi, odp, odt,
                 czp, gzp, czt, gzt,
                 sidx_a, sidx_b, phi_a, phi_b,
                 totals, orow, sem):
    cid = lax.axis_index("c")
    sid = lax.axis_index("s")

    @pl.when(cid == 0)
    def _compute():
        pltpu.sync_copy(tot, totals)
        offp_v = jnp.zeros((16,), f32)
        offt_v = jnp.zeros((16,), f32)
        for r in range(16):
            flag = jnp.where(r < sid, f32(1.0), f32(0.0))
            offp_v = offp_v + totals[r, pl.ds(0, 16)] * flag
            offt_v = offt_v + totals[r, pl.ds(16, 16)] * flag
        offp = jnp.sum(offp_v)
        offt = jnp.sum(offt_v)

        pltpu.sync_copy(cph.at[pl.ds(sid * CP, CP)], czp)
        pltpu.sync_copy(gph.at[pl.ds(sid * CP, CP)], gzp)
        pltpu.sync_copy(cth.at[pl.ds(sid * CT, CT)], czt)
        pltpu.sync_copy(gth.at[pl.ds(sid * CT, CT)], gzt)

        _prefix_and_index(czp, sidx_a, sidx_b, CP, offp)
        _gather_phi(phi, sidx_a, phi_a, CP, sem)
        _gather_phi(phi, sidx_b, phi_b, CP, sem)
        orow[...] = _dcg_accum(czp, gzp, phi_a, phi_b, CP)
        pltpu.sync_copy(orow, odp.at[sid])

        _prefix_and_index(czt, sidx_a, sidx_b, CT, offt)
        _gather_phi(phi, sidx_a, phi_a, CT, sem)
        _gather_phi(phi, sidx_b, phi_b, CT, sem)
        orow[...] = _dcg_accum(czt, gzt, phi_a, phi_b, CT)
        pltpu.sync_copy(orow, odt.at[sid])


ROWS = N // 128
BROWS = 512
GRID = ROWS // BROWS


def _final_body(p_ref, t_ref, dp_ref, dt_ref, out_ref, acc_ref):
    i = pl.program_id(0)

    @pl.when(i == 0)
    def _():
        acc_ref[0] = f32(0.0)

    x = p_ref[...]
    t = t_ref[...]
    bce = jnp.sum(jnp.maximum(x, 0.0) - x * t + jnp.log1p(jnp.exp(-jnp.abs(x))))
    acc_ref[0] += bce

    @pl.when(i == GRID - 1)
    def _():
        dcg = jnp.sum(dp_ref[...])
        ideal = jnp.sum(dt_ref[...])
        xe = acc_ref[0] / f32(N)
        ndcg = dcg / (ideal + f32(1e-8))
        out_ref[0, 0] = xe * (1.0 - ndcg)


_final_call = pl.pallas_call(
    _final_body,
    grid=(GRID,),
    in_specs=[
        pl.BlockSpec((BROWS, 128), lambda i: (i, 0)),
        pl.BlockSpec((BROWS, 128), lambda i: (i, 0)),
        pl.BlockSpec((16, 16), lambda i: (0, 0)),
        pl.BlockSpec((16, 16), lambda i: (0, 0)),
    ],
    out_specs=pl.BlockSpec(memory_space=pltpu.SMEM),
    out_shape=jax.ShapeDtypeStruct((1, 1), f32),
    scratch_shapes=[pltpu.SMEM((1,), f32)],
)


def kernel(predictions, targets):
    pbits = lax.bitcast_convert_type(predictions, i32)
    cp, gp, ct, gt = _hist_kernel(pbits, targets)
    cpm, gpm, ctm, gtm, tot = _merge_kernel(cp, gp, ct, gt)
    dp, dt = _rank_kernel(cpm, gpm, ctm, gtm, tot, _PHI)
    out = _final_call(predictions.reshape(ROWS, 128),
                      targets.reshape(ROWS, 128), dp, dt)
    return out.reshape(())


# speedup vs baseline: 32.7081x; 1.1328x over previous
import functools

import jax
import jax.numpy as jnp
import numpy as np
from jax import lax
from jax.experimental import pallas as pl
from jax.experimental.pallas import tpu as pltpu
from jax.experimental.pallas import tpu_sc as plsc

N = 3276800
NTILES = 32
PER_TILE = N // NTILES
PIECE = 2048
NPIECES = PER_TILE // PIECE

BP_BITS = 14
BP = 1 << BP_BITS
BT = 1 << 15
CP = BP // 16
CT = BT // 16

LN2 = 0.6931471805599453

_f = 1.0 / np.log2(np.arange(N, dtype=np.float64) + 2.0)
_phi = np.zeros(N + 16, dtype=np.float64)
_phi[1:N + 1] = np.cumsum(_f)
_phi[N + 1:] = _phi[N]
_PHI = _phi.astype(np.float32)

_mesh = plsc.VectorSubcoreMesh(core_axis_name="c", subcore_axis_name="s")

f32 = jnp.float32
i32 = jnp.int32


def _zero_ref(ref, size):
    z = jnp.zeros((16,), f32)

    def body(i, _):
        ref[pl.ds(i * 16, 16)] = z
        return 0

    lax.fori_loop(0, size // 16, body, 0, unroll=4)


@functools.partial(
    pl.kernel,
    out_type=(
        jax.ShapeDtypeStruct((NTILES, BP), f32),
        jax.ShapeDtypeStruct((NTILES, BP), f32),
        jax.ShapeDtypeStruct((NTILES, BT), f32),
        jax.ShapeDtypeStruct((NTILES, BT), f32),
    ),
    mesh=_mesh,
    scratch_types=[
        pltpu.VMEM((BP,), f32),
        pltpu.VMEM((BP,), f32),
        pltpu.VMEM((BT,), f32),
        pltpu.VMEM((BT,), f32),
        pltpu.VMEM((2, PIECE), i32),
        pltpu.VMEM((2, PIECE), f32),
        pltpu.SemaphoreType.DMA,
        pltpu.SemaphoreType.DMA,
    ],
    compiler_params=pltpu.CompilerParams(needs_layout_passes=False),
)
def _hist_kernel(pbits, targs, ocp, ogp, oct_, ogt, hcp, hgp, hct, hgt,
                 pbuf, tbuf, sem0, sem1):
    cid = lax.axis_index("c")
    sid = lax.axis_index("s")
    wid = sid * 2 + cid
    base = wid * PER_TILE
    sems = (sem0, sem1)

    _zero_ref(hcp, BP)
    _zero_ref(hgp, BP)
    _zero_ref(hct, BT)
    _zero_ref(hgt, BT)

    ones = jnp.ones((16,), f32)

    def start(pi, b):
        off = base + pi * PIECE
        pltpu.async_copy(pbits.at[pl.ds(off, PIECE)], pbuf.at[b], sems[b])
        pltpu.async_copy(targs.at[pl.ds(off, PIECE)], tbuf.at[b], sems[b])

    start(0, 0)
    start(1, 1)

    def super_body(si, _):
        for b in range(2):
            pi = si * 2 + b
            pltpu.make_async_copy(pbits.at[pl.ds(0, PIECE)], pbuf.at[b],
                                  sems[b]).wait()
            pltpu.make_async_copy(targs.at[pl.ds(0, PIECE)], tbuf.at[b],
                                  sems[b]).wait()

            def vec_body(vi, _, b=b):
                u = pbuf[b, pl.ds(vi * 16, 16)]
                t = tbuf[b, pl.ds(vi * 16, 16)]
                dkey = jnp.where(u < 0, u,
                                 jnp.bitwise_and(jnp.bitwise_not(u),
                                                 jnp.int32(0x7FFFFFFF)))
                bp_idx = lax.shift_right_logical(dkey, 32 - BP_BITS)
                ti = jnp.clip((t * f32(BT)).astype(i32), 0, BT - 1)
                bt_idx = (BT - 1) - ti
                g = jnp.exp(t * f32(LN2)) - 1.0
                plsc.addupdate_scatter(hcp, [bp_idx], ones)
                plsc.addupdate_scatter(hgp, [bp_idx], g)
                plsc.addupdate_scatter(hct, [bt_idx], ones)
                plsc.addupdate_scatter(hgt, [bt_idx], g)
                return 0

            lax.fori_loop(0, PIECE // 16, vec_body, 0, unroll=4)

            @pl.when(pi + 2 < NPIECES)
            def _(pi=pi, b=b):
                start(pi + 2, b)
        return 0

    lax.fori_loop(0, NPIECES // 2, super_body, 0)

    pltpu.sync_copy(hcp, ocp.at[wid])
    pltpu.sync_copy(hgp, ogp.at[wid])
    pltpu.sync_copy(hct, oct_.at[wid])
    pltpu.sync_copy(hgt, ogt.at[wid])


def _reduce_rows(buf, acc, width):
    def body(vi, _):
        s = jnp.zeros((16,), f32)
        for r in range(NTILES):
            s = s + buf[r, pl.ds(vi * 16, 16)]
        acc[pl.ds(vi * 16, 16)] = s
        return 0

    lax.fori_loop(0, width // 16, body, 0)


def _vec_total(acc, width):
    def body(vi, s):
        return s + acc[pl.ds(vi * 16, 16)]

    return lax.fori_loop(0, width // 16, body, jnp.zeros((16,), f32))


def _prefix_and_index(acc_c, sidx_a, sidx_b, width, off):
    def body(vi, carry):
        c = acc_c[pl.ds(vi * 16, 16)]
        inc = plsc.cumsum(c)
        excl = inc - c + carry
        sidx_a[pl.ds(vi * 16, 16)] = excl.astype(i32)
        sidx_b[pl.ds(vi * 16, 16)] = (excl + c).astype(i32)
        return carry + jnp.sum(c)

    lax.fori_loop(0, width // 16, body, off)


def _fire_gather_phi(phi, sidx, dst, width, sem):
    return [pltpu.async_copy(phi.at[sidx.at[pl.ds(j * 128, 128)]],
                             dst.at[pl.ds(j * 128, 128)], sem)
            for j in range(width // 128)]


def _dcg_accum(acc_c, acc_g, phi_a, phi_b, width):
    def body(vi, s):
        sl = pl.ds(vi * 16, 16)
        c = acc_c[sl]
        g = acc_g[sl]
        w = (phi_b[sl] - phi_a[sl]) / jnp.maximum(c, 1.0)
        return s + g * w

    return lax.fori_loop(0, width // 16, body, jnp.zeros((16,), f32))


@functools.partial(
    pl.kernel,
    out_type=(
        jax.ShapeDtypeStruct((BP,), f32),
        jax.ShapeDtypeStruct((BP,), f32),
        jax.ShapeDtypeStruct((BT,), f32),
        jax.ShapeDtypeStruct((BT,), f32),
        jax.ShapeDtypeStruct((16, 32), f32),
    ),
    mesh=_mesh,
    scratch_types=[
        pltpu.VMEM((NTILES, CP), f32),
        pltpu.VMEM((NTILES, CT), f32),
        pltpu.VMEM((CP,), f32),
        pltpu.VMEM((CP,), f32),
        pltpu.VMEM((CT,), f32),
        pltpu.VMEM((CT,), f32),
        pltpu.VMEM((32,), f32),
    ],
    compiler_params=pltpu.CompilerParams(needs_layout_passes=False),
)
def _merge_kernel(hcp, hgp, hct, hgt, ocp, ogp, oct_, ogt, otot,
                  buf_p, buf_t, czp, gzp, czt, gzt, pub):
    cid = lax.axis_index("c")
    sid = lax.axis_index("s")

    @pl.when(cid == 0)
    def _stage():
        pltpu.sync_copy(hcp.at[:, pl.ds(sid * CP, CP)], buf_p)
        _reduce_rows(buf_p, czp, CP)
        pltpu.sync_copy(hgp.at[:, pl.ds(sid * CP, CP)], buf_p)
        _reduce_rows(buf_p, gzp, CP)
        pltpu.sync_copy(hct.at[:, pl.ds(sid * CT, CT)], buf_t)
        _reduce_rows(buf_t, czt, CT)
        pltpu.sync_copy(hgt.at[:, pl.ds(sid * CT, CT)], buf_t)
        _reduce_rows(buf_t, gzt, CT)
        pltpu.sync_copy(czp, ocp.at[pl.ds(sid * CP, CP)])
        pltpu.sync_copy(gzp, ogp.at[pl.ds(sid * CP, CP)])
        pltpu.sync_copy(czt, oct_.at[pl.ds(sid * CT, CT)])
        pltpu.sync_copy(gzt, ogt.at[pl.ds(sid * CT, CT)])
        pub[pl.ds(0, 16)] = _vec_total(czp, CP)
        pub[pl.ds(16, 16)] = _vec_total(czt, CT)
        pltpu.sync_copy(pub, otot.at[sid])


@functools.partial(
    pl.kernel,
    out_type=(
        jax.ShapeDtypeStruct((16, 16), f32),
        jax.ShapeDtypeStruct((16, 16), f32),
    ),
    mesh=_mesh,
    scratch_types=[
        pltpu.VMEM((CP,), f32),
        pltpu.VMEM((CP,), f32),
        pltpu.VMEM((CT,), f32),
        pltpu.VMEM((CT,), f32),
        pltpu.VMEM((CP,), i32),
        pltpu.VMEM((CP,), i32),
        pltpu.VMEM((CT,), i32),
        pltpu.VMEM((CT,), i32),
        pltpu.VMEM((CP,), f32),
        pltpu.VMEM((CP,), f32),
        pltpu.VMEM((CT,), f32),
        pltpu.VMEM((CT,), f32),
        pltpu.VMEM((16, 32), f32),
        pltpu.VMEM((16,), f32),
        pltpu.SemaphoreType.DMA,
    ],
    compiler_params=pltpu.CompilerParams(needs_layout_passes=False),
)
def _rank_kernel(cph, gph, cth, gth, tot, phi, odp, odt,
                 czp, gzp, czt, gzt,
                 sidx_pa, sidx_pb, sidx_ta, sidx_tb,
                 phi_pa, phi_pb, phi_ta, phi_tb,
                 totals, orow, sem):
    cid = lax.axis_index("c")
    sid = lax.axis_index("s")

    @pl.when(cid == 0)
    def _compute():
        pltpu.sync_copy(tot, totals)
        offp_v = jnp.zeros((16,), f32)
        offt_v = jnp.zeros((16,), f32)
        for r in range(16):
            flag = jnp.where(r < sid, f32(1.0), f32(0.0))
            offp_v = offp_v + totals[r, pl.ds(0, 16)] * flag
            offt_v = offt_v + totals[r, pl.ds(16, 16)] * flag
        offp = jnp.sum(offp_v)
        offt = jnp.sum(offt_v)

        loads = [
            pltpu.async_copy(cph.at[pl.ds(sid * CP, CP)], czp, sem),
            pltpu.async_copy(gph.at[pl.ds(sid * CP, CP)], gzp, sem),
            pltpu.async_copy(cth.at[pl.ds(sid * CT, CT)], czt, sem),
            pltpu.async_copy(gth.at[pl.ds(sid * CT, CT)], gzt, sem),
        ]
        for c in loads:
            c.wait()

        _prefix_and_index(czp, sidx_pa, sidx_pb, CP, offp)
        _prefix_and_index(czt, sidx_ta, sidx_tb, CT, offt)
        copies = (
            _fire_gather_phi(phi, sidx_pa, phi_pa, CP, sem)
            + _fire_gather_phi(phi, sidx_pb, phi_pb, CP, sem)
            + _fire_gather_phi(phi, sidx_ta, phi_ta, CT, sem)
            + _fire_gather_phi(phi, sidx_tb, phi_tb, CT, sem)
        )
        for c in copies:
            c.wait()

        orow[...] = _dcg_accum(czp, gzp, phi_pa, phi_pb, CP)
        pltpu.sync_copy(orow, odp.at[sid])
        orow[...] = _dcg_accum(czt, gzt, phi_ta, phi_tb, CT)
        pltpu.sync_copy(orow, odt.at[sid])


ROWS = N // 128
BROWS = 512
GRID = ROWS // BROWS


def _final_body(p_ref, t_ref, dp_ref, dt_ref, out_ref, acc_ref):
    i = pl.program_id(0)

    @pl.when(i == 0)
    def _():
        acc_ref[0] = f32(0.0)

    x = p_ref[...]
    t = t_ref[...]
    bce = jnp.sum(jnp.maximum(x, 0.0) - x * t + jnp.log1p(jnp.exp(-jnp.abs(x))))
    acc_ref[0] += bce

    @pl.when(i == GRID - 1)
    def _():
        dcg = jnp.sum(dp_ref[...])
        ideal = jnp.sum(dt_ref[...])
        xe = acc_ref[0] / f32(N)
        ndcg = dcg / (ideal + f32(1e-8))
        out_ref[0, 0] = xe * (1.0 - ndcg)


_final_call = pl.pallas_call(
    _final_body,
    grid=(GRID,),
    in_specs=[
        pl.BlockSpec((BROWS, 128), lambda i: (i, 0)),
        pl.BlockSpec((BROWS, 128), lambda i: (i, 0)),
        pl.BlockSpec((16, 16), lambda i: (0, 0)),
        pl.BlockSpec((16, 16), lambda i: (0, 0)),
    ],
    out_specs=pl.BlockSpec(memory_space=pltpu.SMEM),
    out_shape=jax.ShapeDtypeStruct((1, 1), f32),
    scratch_shapes=[pltpu.SMEM((1,), f32)],
)


def kernel(predictions, targets):
    pbits = lax.bitcast_convert_type(predictions, i32)
    cp, gp, ct, gt = _hist_kernel(pbits, targets)
    cpm, gpm, ctm, gtm, tot = _merge_kernel(cp, gp, ct, gt)
    dp, dt = _rank_kernel(cpm, gpm, ctm, gtm, tot, _PHI)
    out = _final_call(predictions.reshape(ROWS, 128),
                      targets.reshape(ROWS, 128), dp, dt)
    return out.reshape(())


# speedup vs baseline: 43.0747x; 1.3169x over previous
import functools

import jax
import jax.numpy as jnp
import numpy as np
from jax import lax
from jax.experimental import pallas as pl
from jax.experimental.pallas import tpu as pltpu
from jax.experimental.pallas import tpu_sc as plsc

N = 3276800
NTILES = 32
PER_TILE = N // NTILES
PIECE = 2048
NPIECES = PER_TILE // PIECE

BP_BITS = 14
BP = 1 << BP_BITS
BT = 1 << 15
CP = BP // 16
CT = BT // 16

LN2 = 0.6931471805599453

_f = 1.0 / np.log2(np.arange(N, dtype=np.float64) + 2.0)
_phi = np.zeros(N + 16, dtype=np.float64)
_phi[1:N + 1] = np.cumsum(_f)
_phi[N + 1:] = _phi[N]
_PHI = _phi.astype(np.float32)

_mesh = plsc.VectorSubcoreMesh(core_axis_name="c", subcore_axis_name="s")

f32 = jnp.float32
i32 = jnp.int32


def _zero_ref(ref, size):
    z = jnp.zeros((16,), f32)

    def body(i, _):
        ref[pl.ds(i * 16, 16)] = z
        return 0

    lax.fori_loop(0, size // 16, body, 0, unroll=4)


@functools.partial(
    pl.kernel,
    out_type=(
        jax.ShapeDtypeStruct((NTILES, BP), f32),
        jax.ShapeDtypeStruct((NTILES, BP), f32),
        jax.ShapeDtypeStruct((NTILES, BT), f32),
        jax.ShapeDtypeStruct((NTILES, BT), f32),
    ),
    mesh=_mesh,
    scratch_types=[
        pltpu.VMEM((BP,), f32),
        pltpu.VMEM((BP,), f32),
        pltpu.VMEM((BT,), f32),
        pltpu.VMEM((BT,), f32),
        pltpu.VMEM((2, PIECE), i32),
        pltpu.VMEM((2, PIECE), f32),
        pltpu.SemaphoreType.DMA,
        pltpu.SemaphoreType.DMA,
    ],
    compiler_params=pltpu.CompilerParams(needs_layout_passes=False),
)
def _hist_kernel(pbits, targs, ocp, ogp, oct_, ogt, hcp, hgp, hct, hgt,
                 pbuf, tbuf, sem0, sem1):
    cid = lax.axis_index("c")
    sid = lax.axis_index("s")
    wid = sid * 2 + cid
    base = wid * PER_TILE
    sems = (sem0, sem1)

    _zero_ref(hcp, BP)
    _zero_ref(hgp, BP)
    _zero_ref(hct, BT)
    _zero_ref(hgt, BT)

    ones = jnp.ones((16,), f32)

    def start(pi, b):
        off = base + pi * PIECE
        pltpu.async_copy(pbits.at[pl.ds(off, PIECE)], pbuf.at[b], sems[b])
        pltpu.async_copy(targs.at[pl.ds(off, PIECE)], tbuf.at[b], sems[b])

    start(0, 0)
    start(1, 1)

    def super_body(si, _):
        for b in range(2):
            pi = si * 2 + b
            pltpu.make_async_copy(pbits.at[pl.ds(0, PIECE)], pbuf.at[b],
                                  sems[b]).wait()
            pltpu.make_async_copy(targs.at[pl.ds(0, PIECE)], tbuf.at[b],
                                  sems[b]).wait()

            @plsc.parallel_loop(0, PIECE // 16, unroll=4)
            def vec_body(vi, b=b):
                u = pbuf[b, pl.ds(vi * 16, 16)]
                t = tbuf[b, pl.ds(vi * 16, 16)]
                dkey = jnp.where(u < 0, u,
                                 jnp.bitwise_and(jnp.bitwise_not(u),
                                                 jnp.int32(0x7FFFFFFF)))
                bp_idx = lax.shift_right_logical(dkey, 32 - BP_BITS)
                ti = jnp.clip((t * f32(BT)).astype(i32), 0, BT - 1)
                bt_idx = (BT - 1) - ti
                g = jnp.exp(t * f32(LN2)) - 1.0
                plsc.addupdate_scatter(hcp, [bp_idx], ones)
                plsc.addupdate_scatter(hgp, [bp_idx], g)
                plsc.addupdate_scatter(hct, [bt_idx], ones)
                plsc.addupdate_scatter(hgt, [bt_idx], g)

            @pl.when(pi + 2 < NPIECES)
            def _(pi=pi, b=b):
                start(pi + 2, b)
        return 0

    lax.fori_loop(0, NPIECES // 2, super_body, 0)

    pltpu.sync_copy(hcp, ocp.at[wid])
    pltpu.sync_copy(hgp, ogp.at[wid])
    pltpu.sync_copy(hct, oct_.at[wid])
    pltpu.sync_copy(hgt, ogt.at[wid])


def _reduce_rows(buf, acc, width):
    def body(vi, _):
        s = jnp.zeros((16,), f32)
        for r in range(NTILES):
            s = s + buf[r, pl.ds(vi * 16, 16)]
        acc[pl.ds(vi * 16, 16)] = s
        return 0

    lax.fori_loop(0, width // 16, body, 0)


def _vec_total(acc, width):
    def body(vi, s):
        return s + acc[pl.ds(vi * 16, 16)]

    return lax.fori_loop(0, width // 16, body, jnp.zeros((16,), f32))


def _prefix_and_index(acc_c, sidx_a, sidx_b, width, off):
    def body(vi, carry):
        c = acc_c[pl.ds(vi * 16, 16)]
        inc = plsc.cumsum(c)
        excl = inc - c + carry
        sidx_a[pl.ds(vi * 16, 16)] = excl.astype(i32)
        sidx_b[pl.ds(vi * 16, 16)] = (excl + c).astype(i32)
        return carry + jnp.sum(c)

    lax.fori_loop(0, width // 16, body, off)


def _fire_gather_phi(phi, sidx, dst, width, sem):
    return [pltpu.async_copy(phi.at[sidx.at[pl.ds(j * 128, 128)]],
                             dst.at[pl.ds(j * 128, 128)], sem)
            for j in range(width // 128)]


def _dcg_accum(acc_c, acc_g, phi_a, phi_b, width):
    def body(vi, s):
        sl = pl.ds(vi * 16, 16)
        c = acc_c[sl]
        g = acc_g[sl]
        w = (phi_b[sl] - phi_a[sl]) / jnp.maximum(c, 1.0)
        return s + g * w

    return lax.fori_loop(0, width // 16, body, jnp.zeros((16,), f32))


@functools.partial(
    pl.kernel,
    out_type=(
        jax.ShapeDtypeStruct((BP,), f32),
        jax.ShapeDtypeStruct((BP,), f32),
        jax.ShapeDtypeStruct((BT,), f32),
        jax.ShapeDtypeStruct((BT,), f32),
        jax.ShapeDtypeStruct((16, 32), f32),
    ),
    mesh=_mesh,
    scratch_types=[
        pltpu.VMEM((NTILES, CP), f32),
        pltpu.VMEM((NTILES, CT), f32),
        pltpu.VMEM((CP,), f32),
        pltpu.VMEM((CP,), f32),
        pltpu.VMEM((CT,), f32),
        pltpu.VMEM((CT,), f32),
        pltpu.VMEM((32,), f32),
    ],
    compiler_params=pltpu.CompilerParams(needs_layout_passes=False),
)
def _merge_kernel(hcp, hgp, hct, hgt, ocp, ogp, oct_, ogt, otot,
                  buf_p, buf_t, czp, gzp, czt, gzt, pub):
    cid = lax.axis_index("c")
    sid = lax.axis_index("s")

    @pl.when(cid == 0)
    def _stage():
        pltpu.sync_copy(hcp.at[:, pl.ds(sid * CP, CP)], buf_p)
        _reduce_rows(buf_p, czp, CP)
        pltpu.sync_copy(hgp.at[:, pl.ds(sid * CP, CP)], buf_p)
        _reduce_rows(buf_p, gzp, CP)
        pltpu.sync_copy(hct.at[:, pl.ds(sid * CT, CT)], buf_t)
        _reduce_rows(buf_t, czt, CT)
        pltpu.sync_copy(hgt.at[:, pl.ds(sid * CT, CT)], buf_t)
        _reduce_rows(buf_t, gzt, CT)
        pltpu.sync_copy(czp, ocp.at[pl.ds(sid * CP, CP)])
        pltpu.sync_copy(gzp, ogp.at[pl.ds(sid * CP, CP)])
        pltpu.sync_copy(czt, oct_.at[pl.ds(sid * CT, CT)])
        pltpu.sync_copy(gzt, ogt.at[pl.ds(sid * CT, CT)])
        pub[pl.ds(0, 16)] = _vec_total(czp, CP)
        pub[pl.ds(16, 16)] = _vec_total(czt, CT)
        pltpu.sync_copy(pub, otot.at[sid])


@functools.partial(
    pl.kernel,
    out_type=(
        jax.ShapeDtypeStruct((16, 16), f32),
        jax.ShapeDtypeStruct((16, 16), f32),
    ),
    mesh=_mesh,
    scratch_types=[
        pltpu.VMEM((CP,), f32),
        pltpu.VMEM((CP,), f32),
        pltpu.VMEM((CT,), f32),
        pltpu.VMEM((CT,), f32),
        pltpu.VMEM((CP,), i32),
        pltpu.VMEM((CP,), i32),
        pltpu.VMEM((CT,), i32),
        pltpu.VMEM((CT,), i32),
        pltpu.VMEM((CP,), f32),
        pltpu.VMEM((CP,), f32),
        pltpu.VMEM((CT,), f32),
        pltpu.VMEM((CT,), f32),
        pltpu.VMEM((16, 32), f32),
        pltpu.VMEM((16,), f32),
        pltpu.SemaphoreType.DMA,
    ],
    compiler_params=pltpu.CompilerParams(needs_layout_passes=False),
)
def _rank_kernel(cph, gph, cth, gth, tot, phi, odp, odt,
                 czp, gzp, czt, gzt,
                 sidx_pa, sidx_pb, sidx_ta, sidx_tb,
                 phi_pa, phi_pb, phi_ta, phi_tb,
                 totals, orow, sem):
    cid = lax.axis_index("c")
    sid = lax.axis_index("s")

    @pl.when(cid == 0)
    def _compute():
        pltpu.sync_copy(tot, totals)
        offp_v = jnp.zeros((16,), f32)
        offt_v = jnp.zeros((16,), f32)
        for r in range(16):
            flag = jnp.where(r < sid, f32(1.0), f32(0.0))
            offp_v = offp_v + totals[r, pl.ds(0, 16)] * flag
            offt_v = offt_v + totals[r, pl.ds(16, 16)] * flag
        offp = jnp.sum(offp_v)
        offt = jnp.sum(offt_v)

        loads = [
            pltpu.async_copy(cph.at[pl.ds(sid * CP, CP)], czp, sem),
            pltpu.async_copy(gph.at[pl.ds(sid * CP, CP)], gzp, sem),
            pltpu.async_copy(cth.at[pl.ds(sid * CT, CT)], czt, sem),
            pltpu.async_copy(gth.at[pl.ds(sid * CT, CT)], gzt, sem),
        ]
        for c in loads:
            c.wait()

        _prefix_and_index(czp, sidx_pa, sidx_pb, CP, offp)
        _prefix_and_index(czt, sidx_ta, sidx_tb, CT, offt)
        copies = (
            _fire_gather_phi(phi, sidx_pa, phi_pa, CP, sem)
            + _fire_gather_phi(phi, sidx_pb, phi_pb, CP, sem)
            + _fire_gather_phi(phi, sidx_ta, phi_ta, CT, sem)
            + _fire_gather_phi(phi, sidx_tb, phi_tb, CT, sem)
        )
        for c in copies:
            c.wait()

        orow[...] = _dcg_accum(czp, gzp, phi_pa, phi_pb, CP)
        pltpu.sync_copy(orow, odp.at[sid])
        orow[...] = _dcg_accum(czt, gzt, phi_ta, phi_tb, CT)
        pltpu.sync_copy(orow, odt.at[sid])


ROWS = N // 128
BROWS = 512
GRID = ROWS // BROWS


def _final_body(p_ref, t_ref, dp_ref, dt_ref, out_ref, acc_ref):
    i = pl.program_id(0)

    @pl.when(i == 0)
    def _():
        acc_ref[0] = f32(0.0)

    x = p_ref[...]
    t = t_ref[...]
    bce = jnp.sum(jnp.maximum(x, 0.0) - x * t + jnp.log1p(jnp.exp(-jnp.abs(x))))
    acc_ref[0] += bce

    @pl.when(i == GRID - 1)
    def _():
        dcg = jnp.sum(dp_ref[...])
        ideal = jnp.sum(dt_ref[...])
        xe = acc_ref[0] / f32(N)
        ndcg = dcg / (ideal + f32(1e-8))
        out_ref[0, 0] = xe * (1.0 - ndcg)


_final_call = pl.pallas_call(
    _final_body,
    grid=(GRID,),
    in_specs=[
        pl.BlockSpec((BROWS, 128), lambda i: (i, 0)),
        pl.BlockSpec((BROWS, 128), lambda i: (i, 0)),
        pl.BlockSpec((16, 16), lambda i: (0, 0)),
        pl.BlockSpec((16, 16), lambda i: (0, 0)),
    ],
    out_specs=pl.BlockSpec(memory_space=pltpu.SMEM),
    out_shape=jax.ShapeDtypeStruct((1, 1), f32),
    scratch_shapes=[pltpu.SMEM((1,), f32)],
)


def kernel(predictions, targets):
    pbits = lax.bitcast_convert_type(predictions, i32)
    cp, gp, ct, gt = _hist_kernel(pbits, targets)
    cpm, gpm, ctm, gtm, tot = _merge_kernel(cp, gp, ct, gt)
    dp, dt = _rank_kernel(cpm, gpm, ctm, gtm, tot, _PHI)
    out = _final_call(predictions.reshape(ROWS, 128),
                      targets.reshape(ROWS, 128), dp, dt)
    return out.reshape(())


# speedup vs baseline: 45.4892x; 1.0561x over previous
import functools

import jax
import jax.numpy as jnp
import numpy as np
from jax import lax
from jax.experimental import pallas as pl
from jax.experimental.pallas import tpu as pltpu
from jax.experimental.pallas import tpu_sc as plsc

N = 3276800
NTILES = 32
PER_TILE = N // NTILES
PIECE = 2048
NPIECES = PER_TILE // PIECE

BP_BITS = 14
BP = 1 << BP_BITS
BT = 1 << 15
CP = BP // 32
CT = BT // 32

LN2 = 0.6931471805599453

_f = 1.0 / np.log2(np.arange(N, dtype=np.float64) + 2.0)
_phi = np.zeros(N + 16, dtype=np.float64)
_phi[1:N + 1] = np.cumsum(_f)
_phi[N + 1:] = _phi[N]
_PHI = _phi.astype(np.float32)

_mesh = plsc.VectorSubcoreMesh(core_axis_name="c", subcore_axis_name="s")

f32 = jnp.float32
i32 = jnp.int32


def _zero_ref(ref, size):
    z = jnp.zeros((16,), f32)

    def body(i, _):
        ref[pl.ds(i * 16, 16)] = z
        return 0

    lax.fori_loop(0, size // 16, body, 0, unroll=4)


@functools.partial(
    pl.kernel,
    out_type=(
        jax.ShapeDtypeStruct((NTILES, BP), f32),
        jax.ShapeDtypeStruct((NTILES, BP), f32),
        jax.ShapeDtypeStruct((NTILES, BT), f32),
        jax.ShapeDtypeStruct((NTILES, BT), f32),
    ),
    mesh=_mesh,
    scratch_types=[
        pltpu.VMEM((BP,), f32),
        pltpu.VMEM((BP,), f32),
        pltpu.VMEM((BT,), f32),
        pltpu.VMEM((BT,), f32),
        pltpu.VMEM((2, PIECE), i32),
        pltpu.VMEM((2, PIECE), f32),
        pltpu.SemaphoreType.DMA,
        pltpu.SemaphoreType.DMA,
    ],
    compiler_params=pltpu.CompilerParams(needs_layout_passes=False),
)
def _hist_kernel(pbits, targs, ocp, ogp, oct_, ogt, hcp, hgp, hct, hgt,
                 pbuf, tbuf, sem0, sem1):
    cid = lax.axis_index("c")
    sid = lax.axis_index("s")
    wid = sid * 2 + cid
    base = wid * PER_TILE
    sems = (sem0, sem1)

    _zero_ref(hcp, BP)
    _zero_ref(hgp, BP)
    _zero_ref(hct, BT)
    _zero_ref(hgt, BT)

    ones = jnp.ones((16,), f32)

    def start(pi, b):
        off = base + pi * PIECE
        pltpu.async_copy(pbits.at[pl.ds(off, PIECE)], pbuf.at[b], sems[b])
        pltpu.async_copy(targs.at[pl.ds(off, PIECE)], tbuf.at[b], sems[b])

    start(0, 0)
    start(1, 1)

    def super_body(si, _):
        for b in range(2):
            pi = si * 2 + b
            pltpu.make_async_copy(pbits.at[pl.ds(0, PIECE)], pbuf.at[b],
                                  sems[b]).wait()
            pltpu.make_async_copy(targs.at[pl.ds(0, PIECE)], tbuf.at[b],
                                  sems[b]).wait()

            @plsc.parallel_loop(0, PIECE // 16, unroll=4)
            def vec_body(vi, b=b):
                u = pbuf[b, pl.ds(vi * 16, 16)]
                t = tbuf[b, pl.ds(vi * 16, 16)]
                dkey = jnp.where(u < 0, u,
                                 jnp.bitwise_and(jnp.bitwise_not(u),
                                                 jnp.int32(0x7FFFFFFF)))
                bp_idx = lax.shift_right_logical(dkey, 32 - BP_BITS)
                ti = jnp.clip((t * f32(BT)).astype(i32), 0, BT - 1)
                bt_idx = (BT - 1) - ti
                g = jnp.exp(t * f32(LN2)) - 1.0
                plsc.addupdate_scatter(hcp, [bp_idx], ones)
                plsc.addupdate_scatter(hgp, [bp_idx], g)
                plsc.addupdate_scatter(hct, [bt_idx], ones)
                plsc.addupdate_scatter(hgt, [bt_idx], g)

            @pl.when(pi + 2 < NPIECES)
            def _(pi=pi, b=b):
                start(pi + 2, b)
        return 0

    lax.fori_loop(0, NPIECES // 2, super_body, 0)

    pltpu.sync_copy(hcp, ocp.at[wid])
    pltpu.sync_copy(hgp, ogp.at[wid])
    pltpu.sync_copy(hct, oct_.at[wid])
    pltpu.sync_copy(hgt, ogt.at[wid])


def _reduce_rows(buf, acc, width):
    def body(vi, _):
        s = jnp.zeros((16,), f32)
        for r in range(NTILES):
            s = s + buf[r, pl.ds(vi * 16, 16)]
        acc[pl.ds(vi * 16, 16)] = s
        return 0

    lax.fori_loop(0, width // 16, body, 0)


def _vec_total(acc, width):
    def body(vi, s):
        return s + acc[pl.ds(vi * 16, 16)]

    return lax.fori_loop(0, width // 16, body, jnp.zeros((16,), f32))


def _prefix_and_index(acc_c, sidx_a, sidx_b, width, off):
    def body(vi, carry):
        c = acc_c[pl.ds(vi * 16, 16)]
        inc = plsc.cumsum(c)
        excl = inc - c + carry
        sidx_a[pl.ds(vi * 16, 16)] = excl.astype(i32)
        sidx_b[pl.ds(vi * 16, 16)] = (excl + c).astype(i32)
        return carry + jnp.sum(c)

    lax.fori_loop(0, width // 16, body, off)


def _fire_gather_phi(phi, sidx, dst, width, sem):
    return [pltpu.async_copy(phi.at[sidx.at[pl.ds(j * 128, 128)]],
                             dst.at[pl.ds(j * 128, 128)], sem)
            for j in range(width // 128)]


def _dcg_accum(acc_c, acc_g, phi_a, phi_b, width):
    def body(vi, s):
        sl = pl.ds(vi * 16, 16)
        c = acc_c[sl]
        g = acc_g[sl]
        w = (phi_b[sl] - phi_a[sl]) / jnp.maximum(c, 1.0)
        return s + g * w

    return lax.fori_loop(0, width // 16, body, jnp.zeros((16,), f32))


@functools.partial(
    pl.kernel,
    out_type=(
        jax.ShapeDtypeStruct((BP,), f32),
        jax.ShapeDtypeStruct((BP,), f32),
        jax.ShapeDtypeStruct((BT,), f32),
        jax.ShapeDtypeStruct((BT,), f32),
        jax.ShapeDtypeStruct((32, 32), f32),
    ),
    mesh=_mesh,
    scratch_types=[
        pltpu.VMEM((NTILES, CP), f32),
        pltpu.VMEM((NTILES, CT), f32),
        pltpu.VMEM((CP,), f32),
        pltpu.VMEM((CP,), f32),
        pltpu.VMEM((CT,), f32),
        pltpu.VMEM((CT,), f32),
        pltpu.VMEM((32,), f32),
    ],
    compiler_params=pltpu.CompilerParams(needs_layout_passes=False),
)
def _merge_kernel(hcp, hgp, hct, hgt, ocp, ogp, oct_, ogt, otot,
                  buf_p, buf_t, czp, gzp, czt, gzt, pub):
    cid = lax.axis_index("c")
    sid = lax.axis_index("s")
    wid = sid * 2 + cid

    pltpu.sync_copy(hcp.at[:, pl.ds(wid * CP, CP)], buf_p)
    _reduce_rows(buf_p, czp, CP)
    pltpu.sync_copy(hgp.at[:, pl.ds(wid * CP, CP)], buf_p)
    _reduce_rows(buf_p, gzp, CP)
    pltpu.sync_copy(hct.at[:, pl.ds(wid * CT, CT)], buf_t)
    _reduce_rows(buf_t, czt, CT)
    pltpu.sync_copy(hgt.at[:, pl.ds(wid * CT, CT)], buf_t)
    _reduce_rows(buf_t, gzt, CT)
    pltpu.sync_copy(czp, ocp.at[pl.ds(wid * CP, CP)])
    pltpu.sync_copy(gzp, ogp.at[pl.ds(wid * CP, CP)])
    pltpu.sync_copy(czt, oct_.at[pl.ds(wid * CT, CT)])
    pltpu.sync_copy(gzt, ogt.at[pl.ds(wid * CT, CT)])
    pub[pl.ds(0, 16)] = _vec_total(czp, CP)
    pub[pl.ds(16, 16)] = _vec_total(czt, CT)
    pltpu.sync_copy(pub, otot.at[wid])


@functools.partial(
    pl.kernel,
    out_type=(
        jax.ShapeDtypeStruct((32, 16), f32),
        jax.ShapeDtypeStruct((32, 16), f32),
    ),
    mesh=_mesh,
    scratch_types=[
        pltpu.VMEM((CP,), f32),
        pltpu.VMEM((CP,), f32),
        pltpu.VMEM((CT,), f32),
        pltpu.VMEM((CT,), f32),
        pltpu.VMEM((CP,), i32),
        pltpu.VMEM((CP,), i32),
        pltpu.VMEM((CT,), i32),
        pltpu.VMEM((CT,), i32),
        pltpu.VMEM((CP,), f32),
        pltpu.VMEM((CP,), f32),
        pltpu.VMEM((CT,), f32),
        pltpu.VMEM((CT,), f32),
        pltpu.VMEM((32, 32), f32),
        pltpu.VMEM((16,), f32),
        pltpu.SemaphoreType.DMA,
    ],
    compiler_params=pltpu.CompilerParams(needs_layout_passes=False),
)
def _rank_kernel(cph, gph, cth, gth, tot, phi, odp, odt,
                 czp, gzp, czt, gzt,
                 sidx_pa, sidx_pb, sidx_ta, sidx_tb,
                 phi_pa, phi_pb, phi_ta, phi_tb,
                 totals, orow, sem):
    cid = lax.axis_index("c")
    sid = lax.axis_index("s")
    wid = sid * 2 + cid

    pltpu.sync_copy(tot, totals)
    offp_v = jnp.zeros((16,), f32)
    offt_v = jnp.zeros((16,), f32)
    for r in range(32):
        flag = jnp.where(r < wid, f32(1.0), f32(0.0))
        offp_v = offp_v + totals[r, pl.ds(0, 16)] * flag
        offt_v = offt_v + totals[r, pl.ds(16, 16)] * flag
    offp = jnp.sum(offp_v)
    offt = jnp.sum(offt_v)

    loads = [
        pltpu.async_copy(cph.at[pl.ds(wid * CP, CP)], czp, sem),
        pltpu.async_copy(gph.at[pl.ds(wid * CP, CP)], gzp, sem),
        pltpu.async_copy(cth.at[pl.ds(wid * CT, CT)], czt, sem),
        pltpu.async_copy(gth.at[pl.ds(wid * CT, CT)], gzt, sem),
    ]
    for c in loads:
        c.wait()

    _prefix_and_index(czp, sidx_pa, sidx_pb, CP, offp)
    _prefix_and_index(czt, sidx_ta, sidx_tb, CT, offt)
    copies = (
        _fire_gather_phi(phi, sidx_pa, phi_pa, CP, sem)
        + _fire_gather_phi(phi, sidx_pb, phi_pb, CP, sem)
        + _fire_gather_phi(phi, sidx_ta, phi_ta, CT, sem)
        + _fire_gather_phi(phi, sidx_tb, phi_tb, CT, sem)
    )
    for c in copies:
        c.wait()

    orow[...] = _dcg_accum(czp, gzp, phi_pa, phi_pb, CP)
    pltpu.sync_copy(orow, odp.at[wid])
    orow[...] = _dcg_accum(czt, gzt, phi_ta, phi_tb, CT)
    pltpu.sync_copy(orow, odt.at[wid])


ROWS = N // 128
BROWS = 512
GRID = ROWS // BROWS


def _final_body(p_ref, t_ref, dp_ref, dt_ref, out_ref, acc_ref):
    i = pl.program_id(0)

    @pl.when(i == 0)
    def _():
        acc_ref[0] = f32(0.0)

    x = p_ref[...]
    t = t_ref[...]
    bce = jnp.sum(jnp.maximum(x, 0.0) - x * t + jnp.log1p(jnp.exp(-jnp.abs(x))))
    acc_ref[0] += bce

    @pl.when(i == GRID - 1)
    def _():
        dcg = jnp.sum(dp_ref[...])
        ideal = jnp.sum(dt_ref[...])
        xe = acc_ref[0] / f32(N)
        ndcg = dcg / (ideal + f32(1e-8))
        out_ref[0, 0] = xe * (1.0 - ndcg)


_final_call = pl.pallas_call(
    _final_body,
    grid=(GRID,),
    in_specs=[
        pl.BlockSpec((BROWS, 128), lambda i: (i, 0)),
        pl.BlockSpec((BROWS, 128), lambda i: (i, 0)),
        pl.BlockSpec((32, 16), lambda i: (0, 0)),
        pl.BlockSpec((32, 16), lambda i: (0, 0)),
    ],
    out_specs=pl.BlockSpec(memory_space=pltpu.SMEM),
    out_shape=jax.ShapeDtypeStruct((1, 1), f32),
    scratch_shapes=[pltpu.SMEM((1,), f32)],
)


def kernel(predictions, targets):
    pbits = lax.bitcast_convert_type(predictions, i32)
    cp, gp, ct, gt = _hist_kernel(pbits, targets)
    cpm, gpm, ctm, gtm, tot = _merge_kernel(cp, gp, ct, gt)
    dp, dt = _rank_kernel(cpm, gpm, ctm, gtm, tot, _PHI)
    out = _final_call(predictions.reshape(ROWS, 128),
                      targets.reshape(ROWS, 128), dp, dt)
    return out.reshape(())


# speedup vs baseline: 59.2739x; 1.3030x over previous
import functools

import jax
import jax.numpy as jnp
import numpy as np
from jax import lax
from jax.experimental import pallas as pl
from jax.experimental.pallas import tpu as pltpu
from jax.experimental.pallas import tpu_sc as plsc

N = 3276800
NTILES = 32
PER_TILE = N // NTILES
PIECE = 2048
NPIECES = PER_TILE // PIECE

BP_BITS = 14
BP = 1 << BP_BITS
BT = 1 << 15
CP = BP // 32
CT = BT // 32

LN2 = 0.6931471805599453

_f = 1.0 / np.log2(np.arange(N, dtype=np.float64) + 2.0)
_phi = np.zeros(N + 16, dtype=np.float64)
_phi[1:N + 1] = np.cumsum(_f)
_phi[N + 1:] = _phi[N]
_PHI = _phi.astype(np.float32)

_mesh = plsc.VectorSubcoreMesh(core_axis_name="c", subcore_axis_name="s")

f32 = jnp.float32
i32 = jnp.int32


def _zero_ref(ref, size):
    z = jnp.zeros((16,), f32)

    def body(i, _):
        ref[pl.ds(i * 16, 16)] = z
        return 0

    lax.fori_loop(0, size // 16, body, 0, unroll=4)


@functools.partial(
    pl.kernel,
    out_type=(
        jax.ShapeDtypeStruct((NTILES, BP), f32),
        jax.ShapeDtypeStruct((NTILES, BP), f32),
        jax.ShapeDtypeStruct((NTILES, BT), f32),
        jax.ShapeDtypeStruct((NTILES, BT), f32),
    ),
    mesh=_mesh,
    scratch_types=[
        pltpu.VMEM((BP,), f32),
        pltpu.VMEM((BP,), f32),
        pltpu.VMEM((BT,), f32),
        pltpu.VMEM((BT,), f32),
        pltpu.VMEM((2, PIECE), i32),
        pltpu.VMEM((2, PIECE), f32),
        pltpu.SemaphoreType.DMA,
        pltpu.SemaphoreType.DMA,
    ],
    compiler_params=pltpu.CompilerParams(needs_layout_passes=False),
)
def _hist_kernel(pbits, targs, ocp, ogp, oct_, ogt, hcp, hgp, hct, hgt,
                 pbuf, tbuf, sem0, sem1):
    cid = lax.axis_index("c")
    sid = lax.axis_index("s")
    wid = sid * 2 + cid
    base = wid * PER_TILE
    sems = (sem0, sem1)

    _zero_ref(hcp, BP)
    _zero_ref(hgp, BP)
    _zero_ref(hct, BT)
    _zero_ref(hgt, BT)

    ones = jnp.ones((16,), f32)

    def start(pi, b):
        off = base + pi * PIECE
        pltpu.async_copy(pbits.at[pl.ds(off, PIECE)], pbuf.at[b], sems[b])
        pltpu.async_copy(targs.at[pl.ds(off, PIECE)], tbuf.at[b], sems[b])

    start(0, 0)
    start(1, 1)

    def super_body(si, _):
        for b in range(2):
            pi = si * 2 + b
            pltpu.make_async_copy(pbits.at[pl.ds(0, PIECE)], pbuf.at[b],
                                  sems[b]).wait()
            pltpu.make_async_copy(targs.at[pl.ds(0, PIECE)], tbuf.at[b],
                                  sems[b]).wait()

            @plsc.parallel_loop(0, PIECE // 16, unroll=4)
            def vec_body(vi, b=b):
                u = pbuf[b, pl.ds(vi * 16, 16)]
                t = tbuf[b, pl.ds(vi * 16, 16)]
                dkey = jnp.where(u < 0, u,
                                 jnp.bitwise_and(jnp.bitwise_not(u),
                                                 jnp.int32(0x7FFFFFFF)))
                bp_idx = lax.shift_right_logical(dkey, 32 - BP_BITS)
                ti = jnp.clip((t * f32(BT)).astype(i32), 0, BT - 1)
                bt_idx = (BT - 1) - ti
                g = jnp.exp(t * f32(LN2)) - 1.0
                plsc.addupdate_scatter(hcp, [bp_idx], ones)
                plsc.addupdate_scatter(hgp, [bp_idx], g)
                plsc.addupdate_scatter(hct, [bt_idx], ones)
                plsc.addupdate_scatter(hgt, [bt_idx], g)

            @pl.when(pi + 2 < NPIECES)
            def _(pi=pi, b=b):
                start(pi + 2, b)
        return 0

    lax.fori_loop(0, NPIECES // 2, super_body, 0)

    pltpu.sync_copy(hcp, ocp.at[wid])
    pltpu.sync_copy(hgp, ogp.at[wid])
    pltpu.sync_copy(hct, oct_.at[wid])
    pltpu.sync_copy(hgt, ogt.at[wid])


def _reduce_rows(buf, acc, width):
    def body(vi, _):
        s = jnp.zeros((16,), f32)
        for r in range(NTILES):
            s = s + buf[r, pl.ds(vi * 16, 16)]
        acc[pl.ds(vi * 16, 16)] = s
        return 0

    lax.fori_loop(0, width // 16, body, 0)


def _vec_total(acc, width):
    def body(vi, s):
        return s + acc[pl.ds(vi * 16, 16)]

    return lax.fori_loop(0, width // 16, body, jnp.zeros((16,), f32))


def _prefix_and_index(acc_c, sidx_a, width, off):
    def body(vi, carry):
        c = acc_c[pl.ds(vi * 16, 16)]
        inc = plsc.cumsum(c)
        excl = inc - c + carry
        sidx_a[pl.ds(vi * 16, 16)] = excl.astype(i32)
        return carry + jnp.sum(c)

    end = lax.fori_loop(0, width // 16, body, off)
    endv = jnp.full((16,), 1.0, f32) * end
    for j in range(8):
        sidx_a[pl.ds(width + j * 16, 16)] = endv.astype(i32)


def _fire_gather_phi(phi, sidx, dst, width, sem):
    return [pltpu.async_copy(phi.at[sidx.at[pl.ds(j * 128, 128)]],
                             dst.at[pl.ds(j * 128, 128)], sem)
            for j in range(width // 128)]


def _dcg_accum(acc_c, acc_g, phi_a, width):
    def body(vi, s):
        sl = pl.ds(vi * 16, 16)
        c = acc_c[sl]
        g = acc_g[sl]
        pb = phi_a[pl.ds(vi * 16 + 1, 16)]
        w = (pb - phi_a[sl]) / jnp.maximum(c, 1.0)
        return s + g * w

    return lax.fori_loop(0, width // 16, body, jnp.zeros((16,), f32))


@functools.partial(
    pl.kernel,
    out_type=(
        jax.ShapeDtypeStruct((BP,), f32),
        jax.ShapeDtypeStruct((BP,), f32),
        jax.ShapeDtypeStruct((BT,), f32),
        jax.ShapeDtypeStruct((BT,), f32),
        jax.ShapeDtypeStruct((32, 32), f32),
    ),
    mesh=_mesh,
    scratch_types=[
        pltpu.VMEM((NTILES, CP), f32),
        pltpu.VMEM((NTILES, CT), f32),
        pltpu.VMEM((CP,), f32),
        pltpu.VMEM((CP,), f32),
        pltpu.VMEM((CT,), f32),
        pltpu.VMEM((CT,), f32),
        pltpu.VMEM((32,), f32),
    ],
    compiler_params=pltpu.CompilerParams(needs_layout_passes=False),
)
def _merge_kernel(hcp, hgp, hct, hgt, ocp, ogp, oct_, ogt, otot,
                  buf_p, buf_t, czp, gzp, czt, gzt, pub):
    cid = lax.axis_index("c")
    sid = lax.axis_index("s")
    wid = sid * 2 + cid

    pltpu.sync_copy(hcp.at[:, pl.ds(wid * CP, CP)], buf_p)
    _reduce_rows(buf_p, czp, CP)
    pltpu.sync_copy(hgp.at[:, pl.ds(wid * CP, CP)], buf_p)
    _reduce_rows(buf_p, gzp, CP)
    pltpu.sync_copy(hct.at[:, pl.ds(wid * CT, CT)], buf_t)
    _reduce_rows(buf_t, czt, CT)
    pltpu.sync_copy(hgt.at[:, pl.ds(wid * CT, CT)], buf_t)
    _reduce_rows(buf_t, gzt, CT)
    pltpu.sync_copy(czp, ocp.at[pl.ds(wid * CP, CP)])
    pltpu.sync_copy(gzp, ogp.at[pl.ds(wid * CP, CP)])
    pltpu.sync_copy(czt, oct_.at[pl.ds(wid * CT, CT)])
    pltpu.sync_copy(gzt, ogt.at[pl.ds(wid * CT, CT)])
    pub[pl.ds(0, 16)] = _vec_total(czp, CP)
    pub[pl.ds(16, 16)] = _vec_total(czt, CT)
    pltpu.sync_copy(pub, otot.at[wid])


@functools.partial(
    pl.kernel,
    out_type=(
        jax.ShapeDtypeStruct((32, 16), f32),
        jax.ShapeDtypeStruct((32, 16), f32),
    ),
    mesh=_mesh,
    scratch_types=[
        pltpu.VMEM((CP,), f32),
        pltpu.VMEM((CP,), f32),
        pltpu.VMEM((CT,), f32),
        pltpu.VMEM((CT,), f32),
        pltpu.VMEM((CP + 128,), i32),
        pltpu.VMEM((CT + 128,), i32),
        pltpu.VMEM((CP + 128,), f32),
        pltpu.VMEM((CT + 128,), f32),
        pltpu.VMEM((32, 32), f32),
        pltpu.VMEM((16,), f32),
        pltpu.SemaphoreType.DMA,
    ],
    compiler_params=pltpu.CompilerParams(needs_layout_passes=False),
)
def _rank_kernel(cph, gph, cth, gth, tot, phi, odp, odt,
                 czp, gzp, czt, gzt,
                 sidx_pa, sidx_ta, phi_pa, phi_ta,
                 totals, orow, sem):
    cid = lax.axis_index("c")
    sid = lax.axis_index("s")
    wid = sid * 2 + cid

    pltpu.sync_copy(tot, totals)
    offp_v = jnp.zeros((16,), f32)
    offt_v = jnp.zeros((16,), f32)
    for r in range(32):
        flag = jnp.where(r < wid, f32(1.0), f32(0.0))
        offp_v = offp_v + totals[r, pl.ds(0, 16)] * flag
        offt_v = offt_v + totals[r, pl.ds(16, 16)] * flag
    offp = jnp.sum(offp_v)
    offt = jnp.sum(offt_v)

    loads = [
        pltpu.async_copy(cph.at[pl.ds(wid * CP, CP)], czp, sem),
        pltpu.async_copy(gph.at[pl.ds(wid * CP, CP)], gzp, sem),
        pltpu.async_copy(cth.at[pl.ds(wid * CT, CT)], czt, sem),
        pltpu.async_copy(gth.at[pl.ds(wid * CT, CT)], gzt, sem),
    ]
    for c in loads:
        c.wait()

    _prefix_and_index(czp, sidx_pa, CP, offp)
    _prefix_and_index(czt, sidx_ta, CT, offt)
    copies = (
        _fire_gather_phi(phi, sidx_pa, phi_pa, CP + 128, sem)
        + _fire_gather_phi(phi, sidx_ta, phi_ta, CT + 128, sem)
    )
    for c in copies:
        c.wait()

    orow[...] = _dcg_accum(czp, gzp, phi_pa, CP)
    pltpu.sync_copy(orow, odp.at[wid])
    orow[...] = _dcg_accum(czt, gzt, phi_ta, CT)
    pltpu.sync_copy(orow, odt.at[wid])


ROWS = N // 128
BROWS = 512
GRID = ROWS // BROWS


def _bce_body(p_ref, t_ref, out_ref, acc_ref):
    i = pl.program_id(0)

    @pl.when(i == 0)
    def _():
        acc_ref[0] = f32(0.0)

    x = p_ref[...]
    t = t_ref[...]
    bce = jnp.sum(jnp.maximum(x, 0.0) - x * t + jnp.log1p(jnp.exp(-jnp.abs(x))))
    acc_ref[0] += bce

    @pl.when(i == GRID - 1)
    def _():
        out_ref[0, 0] = acc_ref[0]


_bce_call = pl.pallas_call(
    _bce_body,
    grid=(GRID,),
    in_specs=[
        pl.BlockSpec((BROWS, 128), lambda i: (i, 0)),
        pl.BlockSpec((BROWS, 128), lambda i: (i, 0)),
    ],
    out_specs=pl.BlockSpec(memory_space=pltpu.SMEM),
    out_shape=jax.ShapeDtypeStruct((1, 1), f32),
    scratch_shapes=[pltpu.SMEM((1,), f32)],
)


def _combine_body(dp_ref, dt_ref, bce_ref, out_ref):
    dcg = jnp.sum(dp_ref[...])
    ideal = jnp.sum(dt_ref[...])
    xe = bce_ref[0, 0] / f32(N)
    ndcg = dcg / (ideal + f32(1e-8))
    out_ref[0, 0] = xe * (1.0 - ndcg)


_combine_call = pl.pallas_call(
    _combine_body,
    in_specs=[
        pl.BlockSpec((32, 16), lambda: (0, 0)),
        pl.BlockSpec((32, 16), lambda: (0, 0)),
        pl.BlockSpec(memory_space=pltpu.SMEM),
    ],
    out_specs=pl.BlockSpec(memory_space=pltpu.SMEM),
    out_shape=jax.ShapeDtypeStruct((1, 1), f32),
)


def kernel(predictions, targets):
    pbits = lax.bitcast_convert_type(predictions, i32)
    bce = _bce_call(predictions.reshape(ROWS, 128),
                    targets.reshape(ROWS, 128))
    cp, gp, ct, gt = _hist_kernel(pbits, targets)
    cpm, gpm, ctm, gtm, tot = _merge_kernel(cp, gp, ct, gt)
    dp, dt = _rank_kernel(cpm, gpm, ctm, gtm, tot, _PHI)
    out = _combine_call(dp, dt, bce)
    return out.reshape(())
